# baseline (speedup 1.0000x reference)
import jax
import jax.numpy as jnp
from jax import lax
from jax.experimental import pallas as pl
from jax.experimental.pallas import tpu as pltpu
from jax.experimental.pallas import tpu_sc as plsc

N, C = 10000, 128
NC, M = 160000, 512
NPAD, EPAD = 10240, 640
NT = 32
NB, KB = 40, 128
PB = NB * KB
NCPAD = NT * PB

_MESH = plsc.VectorSubcoreMesh(core_axis_name="c", subcore_axis_name="s")


def _wid():
    return lax.axis_index("s") * 2 + lax.axis_index("c")


def _zero16():
    return jnp.zeros((16,), jnp.float32)


def _zero_1d(ref, nwords):
    def body(i, _):
        ref[pl.ds(i * 16, 16)] = _zero16()
        return 0
    lax.fori_loop(0, nwords // 16, body, 0)


def _zero_2d(ref, nrows):
    def body(i, _):
        for j in range(8):
            ref[i, pl.ds(j * 16, 16)] = _zero16()
        return 0
    lax.fori_loop(0, nrows, body, 0)


def _sum_tables(dst, src1, nwords):
    def body(i, _):
        s = pl.ds(i * 16, 16)
        dst[s] = dst[s] + src1[s]
        return 0
    lax.fori_loop(0, nwords // 16, body, 0)


def _core_write(cid, src, dst0, dst1):
    @pl.when(cid == 0)
    def _():
        pltpu.sync_copy(src, dst0)

    @pl.when(cid == 1)
    def _():
        pltpu.sync_copy(src, dst1)


def _tca_body(x_ref, w_ref, a_ref, xp_ref, aux_ref):
    p = jnp.dot(x_ref[...], w_ref[...], preferred_element_type=jnp.float32,
                precision=lax.Precision.HIGHEST)
    xp_ref[...] = p
    aux_ref[...] = lax.dot_general(
        a_ref[...], p, (((0,), (1,)), ((), ())),
        preferred_element_type=jnp.float32,
        precision=lax.Precision.HIGHEST)


def _run_tca(xpad, weight, acat):
    blk = 1024
    return pl.pallas_call(
        _tca_body,
        grid=(NPAD // blk,),
        in_specs=[
            pl.BlockSpec((blk, C), lambda i: (i, 0)),
            pl.BlockSpec((C, C), lambda i: (0, 0)),
            pl.BlockSpec((C, 8), lambda i: (0, 0)),
        ],
        out_specs=[
            pl.BlockSpec((blk, C), lambda i: (i, 0)),
            pl.BlockSpec((8, blk), lambda i: (0, i)),
        ],
        out_shape=[
            jax.ShapeDtypeStruct((NPAD, C), jnp.float32),
            jax.ShapeDtypeStruct((8, NPAD), jnp.float32),
        ],
    )(xpad, weight, acat)


def _sc1_body(node3_hbm, edge3_hbm, nd2_hbm,
              dp0_hbm, dp1_hbm, ep0_hbm, ep1_hbm, tp0_hbm, tp1_hbm,
              idxn, idxe, val, ones_v, nd2_tab, z1d,
              acc_d, acc_e, acc_t,
              sem_a, sem_b, sem_c):
    cid = lax.axis_index("c")
    sid = lax.axis_index("s")
    wid = _wid()

    _zero_1d(z1d, 640)
    pltpu.sync_copy(z1d, acc_d.at[pl.ds(sid * 640, 640)])

    @pl.when(sid == 0)
    def _():
        pltpu.sync_copy(z1d, acc_e)
        pltpu.sync_copy(z1d, acc_t)

    pltpu.sync_copy(node3_hbm.at[wid], idxn)
    pltpu.sync_copy(edge3_hbm.at[wid], idxe)
    pltpu.sync_copy(nd2_hbm, nd2_tab)

    def gbody(b, _):
        for k in range(8):
            s = pl.ds(k * 16, 16)
            val[b, s] = plsc.load_gather(nd2_tab, [idxn[b, s]])
            ones_v[b, s] = jnp.full((16,), 1.0, jnp.float32)
        return 0
    lax.fori_loop(0, NB, gbody, 0)

    plsc.subcore_barrier()

    def sbody(b, _):
        d1 = pltpu.async_copy(ones_v.at[b], acc_d.at[idxn.at[b]], sem_a,
                              add=True)
        d2 = pltpu.async_copy(ones_v.at[b], acc_e.at[idxe.at[b]], sem_b,
                              add=True)
        d3 = pltpu.async_copy(val.at[b], acc_t.at[idxe.at[b]], sem_c,
                              add=True)
        d1.wait()
        d2.wait()
        d3.wait()
        return 0
    lax.fori_loop(0, NB, sbody, 0)

    plsc.subcore_barrier()

    _core_write(cid, acc_d.at[pl.ds(sid * 640, 640)],
                dp0_hbm.at[pl.ds(sid * 640, 640)],
                dp1_hbm.at[pl.ds(sid * 640, 640)])

    @pl.when(sid == 0)
    def _():
        _core_write(cid, acc_e, ep0_hbm, ep1_hbm)
        _core_write(cid, acc_t, tp0_hbm, tp1_hbm)


def _run_sc1(node3, edge3, nd2):
    fn = pl.kernel(
        _sc1_body,
        out_type=[
            jax.ShapeDtypeStruct((NPAD,), jnp.float32),
            jax.ShapeDtypeStruct((NPAD,), jnp.float32),
            jax.ShapeDtypeStruct((EPAD,), jnp.float32),
            jax.ShapeDtypeStruct((EPAD,), jnp.float32),
            jax.ShapeDtypeStruct((EPAD,), jnp.float32),
            jax.ShapeDtypeStruct((EPAD,), jnp.float32),
        ],
        mesh=_MESH,
        compiler_params=pltpu.CompilerParams(needs_layout_passes=False),
        scratch_types=[
            pltpu.VMEM((NB, KB), jnp.int32),
            pltpu.VMEM((NB, KB), jnp.int32),
            pltpu.VMEM((NB, KB), jnp.float32),
            pltpu.VMEM((NB, KB), jnp.float32),
            pltpu.VMEM((NPAD,), jnp.float32),
            pltpu.VMEM((640,), jnp.float32),
            pltpu.VMEM_SHARED((NPAD,), jnp.float32),
            pltpu.VMEM_SHARED((EPAD,), jnp.float32),
            pltpu.VMEM_SHARED((EPAD,), jnp.float32),
            pltpu.SemaphoreType.DMA,
            pltpu.SemaphoreType.DMA,
            pltpu.SemaphoreType.DMA,
        ],
    )
    return fn(node3, edge3, nd2)


def _sc2_body(node3_hbm, edge3_hbm, tp0_hbm, tp1_hbm,
              d2p0_hbm, d2p1_hbm,
              idxn, idxe, val, t0, t1, z1d, acc,
              sem_a):
    cid = lax.axis_index("c")
    sid = lax.axis_index("s")
    wid = _wid()

    _zero_1d(z1d, 640)
    pltpu.sync_copy(z1d, acc.at[pl.ds(sid * 640, 640)])

    pltpu.sync_copy(node3_hbm.at[wid], idxn)
    pltpu.sync_copy(edge3_hbm.at[wid], idxe)
    pltpu.sync_copy(tp0_hbm, t0)
    pltpu.sync_copy(tp1_hbm, t1)
    _sum_tables(t0, t1, EPAD)

    def mbody(i, m):
        return jnp.maximum(m, t0[pl.ds(i * 16, 16)])
    m16 = lax.fori_loop(0, EPAD // 16, mbody,
                        jnp.full((16,), -3.4e38, jnp.float32))
    gmax = jnp.max(m16)

    def gbody(b, _):
        for k in range(8):
            s = pl.ds(k * 16, 16)
            ed = plsc.load_gather(t0, [idxe[b, s]])
            val[b, s] = jnp.exp(ed - gmax)
        return 0
    lax.fori_loop(0, NB, gbody, 0)

    plsc.subcore_barrier()

    def sbody(b, _):
        pltpu.async_copy(val.at[b], acc.at[idxn.at[b]], sem_a,
                         add=True).wait()
        return 0
    lax.fori_loop(0, NB, sbody, 0)

    plsc.subcore_barrier()
    _core_write(cid, acc.at[pl.ds(sid * 640, 640)],
                d2p0_hbm.at[pl.ds(sid * 640, 640)],
                d2p1_hbm.at[pl.ds(sid * 640, 640)])


def _run_sc2(node3, edge3, tp0, tp1):
    fn = pl.kernel(
        _sc2_body,
        out_type=[
            jax.ShapeDtypeStruct((NPAD,), jnp.float32),
            jax.ShapeDtypeStruct((NPAD,), jnp.float32),
        ],
        mesh=_MESH,
        compiler_params=pltpu.CompilerParams(needs_layout_passes=False),
        scratch_types=[
            pltpu.VMEM((NB, KB), jnp.int32),
            pltpu.VMEM((NB, KB), jnp.int32),
            pltpu.VMEM((NB, KB), jnp.float32),
            pltpu.VMEM((EPAD,), jnp.float32),
            pltpu.VMEM((EPAD,), jnp.float32),
            pltpu.VMEM((640,), jnp.float32),
            pltpu.VMEM_SHARED((NPAD,), jnp.float32),
            pltpu.SemaphoreType.DMA,
        ],
    )
    return fn(node3, edge3, tp0, tp1)


def _tcc_body(nd1_ref, d2a_ref, d2b_ref, ta_ref, tb_ref, v_ref):
    ed2 = ta_ref[...] + tb_ref[...]
    gmax = jnp.max(ed2)
    d2 = d2a_ref[...] + d2b_ref[...]
    z = nd1_ref[...] + gmax + jnp.log(jnp.maximum(d2, 1e-38))
    v_ref[...] = jnp.maximum(z, 0.2 * z)


def _run_tcc(nd1_2d, d2p0, d2p1, tp0, tp1):
    return pl.pallas_call(
        _tcc_body,
        out_shape=jax.ShapeDtypeStruct((80, 128), jnp.float32),
    )(nd1_2d, d2p0.reshape(80, 128), d2p1.reshape(80, 128),
      tp0.reshape(5, 128), tp1.reshape(5, 128))


def _sc3_body(node3_hbm, edge3_hbm, nd1_hbm, v_hbm, tp0_hbm, tp1_hbm,
              ex_hbm, dap0_hbm, dap1_hbm,
              idxn, idxe, exbuf, nd1_tab, v_tab, t0, t1, z1d, acc,
              sem_a):
    cid = lax.axis_index("c")
    sid = lax.axis_index("s")
    wid = _wid()

    _zero_1d(z1d, 640)
    pltpu.sync_copy(z1d, acc.at[pl.ds(sid * 640, 640)])

    pltpu.sync_copy(node3_hbm.at[wid], idxn)
    pltpu.sync_copy(edge3_hbm.at[wid], idxe)
    pltpu.sync_copy(nd1_hbm, nd1_tab)
    pltpu.sync_copy(v_hbm, v_tab)
    pltpu.sync_copy(tp0_hbm, t0)
    pltpu.sync_copy(tp1_hbm, t1)
    _sum_tables(t0, t1, EPAD)

    def gbody(b, _):
        for k in range(8):
            s = pl.ds(k * 16, 16)
            ii = idxn[b, s]
            z = plsc.load_gather(nd1_tab, [ii]) + plsc.load_gather(
                t0, [idxe[b, s]])
            e = jnp.maximum(z, 0.2 * z)
            exbuf[b, s] = jnp.exp(e - plsc.load_gather(v_tab, [ii]))
        return 0
    lax.fori_loop(0, NB, gbody, 0)

    pltpu.sync_copy(exbuf, ex_hbm.at[wid])

    plsc.subcore_barrier()

    def sbody(b, _):
        pltpu.async_copy(exbuf.at[b], acc.at[idxn.at[b]], sem_a,
                         add=True).wait()
        return 0
    lax.fori_loop(0, NB, sbody, 0)

    plsc.subcore_barrier()
    _core_write(cid, acc.at[pl.ds(sid * 640, 640)],
                dap0_hbm.at[pl.ds(sid * 640, 640)],
                dap1_hbm.at[pl.ds(sid * 640, 640)])


def _run_sc3(node3, edge3, nd1, v, tp0, tp1):
    fn = pl.kernel(
        _sc3_body,
        out_type=[
            jax.ShapeDtypeStruct((NT, NB, KB), jnp.float32),
            jax.ShapeDtypeStruct((NPAD,), jnp.float32),
            jax.ShapeDtypeStruct((NPAD,), jnp.float32),
        ],
        mesh=_MESH,
        compiler_params=pltpu.CompilerParams(needs_layout_passes=False),
        scratch_types=[
            pltpu.VMEM((NB, KB), jnp.int32),
            pltpu.VMEM((NB, KB), jnp.int32),
            pltpu.VMEM((NB, KB), jnp.float32),
            pltpu.VMEM((NPAD,), jnp.float32),
            pltpu.VMEM((NPAD,), jnp.float32),
            pltpu.VMEM((EPAD,), jnp.float32),
            pltpu.VMEM((EPAD,), jnp.float32),
            pltpu.VMEM((640,), jnp.float32),
            pltpu.VMEM_SHARED((NPAD,), jnp.float32),
            pltpu.SemaphoreType.DMA,
        ],
    )
    return fn(node3, edge3, nd1, v, tp0, tp1)


def _sc4_body(node3_hbm, edge3_hbm, ex_hbm, ra_hbm, bn_hbm, xproj_hbm,
              xepart_hbm, alpha_hbm, espart_hbm,
              idxn, idxe, ab, ra_tab, bn_tab,
              rb0, rb1, sb0, sb1, acc_xe, acc_es,
              sem_g0, sem_g1, sem_s0, sem_s1, sem_u0, sem_u1):
    cid = lax.axis_index("c")
    sid = lax.axis_index("s")
    wid = _wid()

    _zero_2d(rb0, KB)
    pltpu.sync_copy(rb0.at[pl.ds(0, 40)], acc_xe.at[pl.ds(sid * 40, 40)])
    pltpu.sync_copy(rb0.at[pl.ds(0, 40)], acc_es.at[pl.ds(sid * 40, 40)])

    pltpu.sync_copy(node3_hbm.at[wid], idxn)
    pltpu.sync_copy(edge3_hbm.at[wid], idxe)
    pltpu.sync_copy(ex_hbm.at[wid], ab)
    pltpu.sync_copy(ra_hbm, ra_tab)
    pltpu.sync_copy(bn_hbm, bn_tab)

    def abody(b, _):
        for k in range(8):
            s = pl.ds(k * 16, 16)
            ab[b, s] = ab[b, s] * plsc.load_gather(ra_tab, [idxn[b, s]])
        return 0
    lax.fori_loop(0, NB, abody, 0)
    pltpu.sync_copy(ab, alpha_hbm.at[wid])

    def bbody(b, _):
        for k in range(8):
            s = pl.ds(k * 16, 16)
            ab[b, s] = ab[b, s] * plsc.load_gather(bn_tab, [idxe[b, s]])
        return 0
    lax.fori_loop(0, NB, bbody, 0)

    plsc.subcore_barrier()

    bufs = ((rb0, sb0, sem_g0, sem_s0, sem_u0),
            (rb1, sb1, sem_g1, sem_s1, sem_u1))
    pltpu.async_copy(xproj_hbm.at[idxn.at[0]], rb0, sem_g0)
    pltpu.async_copy(xproj_hbm.at[idxn.at[1]], rb1, sem_g1)

    def rbody(bb, _):
        for p in range(2):
            b = bb * 2 + p
            rb, sb, sg, ss, su = bufs[p]
            pltpu.make_async_copy(xproj_hbm.at[idxn.at[b]], rb, sg).wait()

            @pl.when(b >= 2)
            def _():
                pltpu.make_async_copy(sb, acc_xe.at[idxe.at[b]], ss).wait()

            def scale(r, _):
                a16 = plsc.load_gather(ab, [jnp.full((16,), b, jnp.int32),
                                            jnp.full((16,), r, jnp.int32)])
                for j in range(8):
                    s = pl.ds(j * 16, 16)
                    sb[r, s] = rb[r, s] * a16
                return 0
            lax.fori_loop(0, KB, scale, 0)
            un = pltpu.async_copy(rb, acc_es.at[idxe.at[b]], su, add=True)
            pltpu.async_copy(sb, acc_xe.at[idxe.at[b]], ss, add=True)
            un.wait()

            @pl.when(b + 2 < NB)
            def _():
                pltpu.async_copy(xproj_hbm.at[idxn.at[b + 2]], rb, sg)
        return 0
    lax.fori_loop(0, NB // 2, rbody, 0)

    for p in range(2):
        rb, sb, sg, ss, su = bufs[p]
        pltpu.make_async_copy(sb, acc_xe.at[idxe.at[NB - 2 + p]], ss).wait()

    plsc.subcore_barrier()
    pltpu.sync_copy(acc_xe.at[pl.ds(sid * 40, 40)],
                    xepart_hbm.at[cid, pl.ds(sid * 40, 40)])
    pltpu.sync_copy(acc_es.at[pl.ds(sid * 40, 40)],
                    espart_hbm.at[cid, pl.ds(sid * 40, 40)])


def _run_sc4(node3, edge3, ex, ra, bn, xproj):
    fn = pl.kernel(
        _sc4_body,
        out_type=[
            jax.ShapeDtypeStruct((2, EPAD, C), jnp.float32),
            jax.ShapeDtypeStruct((NT, NB, KB), jnp.float32),
            jax.ShapeDtypeStruct((2, EPAD, C), jnp.float32),
        ],
        mesh=_MESH,
        compiler_params=pltpu.CompilerParams(needs_layout_passes=False),
        scratch_types=[
            pltpu.VMEM((NB, KB), jnp.int32),
            pltpu.VMEM((NB, KB), jnp.int32),
            pltpu.VMEM((NB, KB), jnp.float32),
            pltpu.VMEM((NPAD,), jnp.float32),
            pltpu.VMEM((EPAD,), jnp.float32),
            pltpu.VMEM((KB, C), jnp.float32),
            pltpu.VMEM((KB, C), jnp.float32),
            pltpu.VMEM((KB, C), jnp.float32),
            pltpu.VMEM((KB, C), jnp.float32),
            pltpu.VMEM_SHARED((EPAD, C), jnp.float32),
            pltpu.VMEM_SHARED((EPAD, C), jnp.float32),
            pltpu.SemaphoreType.DMA,
            pltpu.SemaphoreType.DMA,
            pltpu.SemaphoreType.DMA,
            pltpu.SemaphoreType.DMA,
            pltpu.SemaphoreType.DMA,
            pltpu.SemaphoreType.DMA,
        ],
    )
    return fn(node3, edge3, ex, ra, bn, xproj)


def _sc5_body(node3_hbm, edge3_hbm, alpha_hbm, xe_hbm,
              opart_hbm,
              idxn, idxe, ab, rb0, rb1, acc_out,
              sem_g0, sem_g1, sem_s0, sem_s1):
    cid = lax.axis_index("c")
    sid = lax.axis_index("s")
    wid = _wid()

    _zero_2d(rb0, KB)
    for kk in range(640 // KB):
        pltpu.sync_copy(rb0, acc_out.at[pl.ds(sid * 640 + kk * KB, KB)])

    pltpu.sync_copy(node3_hbm.at[wid], idxn)
    pltpu.sync_copy(edge3_hbm.at[wid], idxe)
    pltpu.sync_copy(alpha_hbm.at[wid], ab)

    plsc.subcore_barrier()

    bufs = ((rb0, sem_g0, sem_s0), (rb1, sem_g1, sem_s1))
    pltpu.async_copy(xe_hbm.at[idxe.at[0]], rb0, sem_g0)
    pltpu.async_copy(xe_hbm.at[idxe.at[1]], rb1, sem_g1)

    def rbody(bb, _):
        for p in range(2):
            b = bb * 2 + p
            rb, sg, ss = bufs[p]
            pltpu.make_async_copy(xe_hbm.at[idxe.at[b]], rb, sg).wait()

            def scale(r, _):
                a16 = plsc.load_gather(ab, [jnp.full((16,), b, jnp.int32),
                                            jnp.full((16,), r, jnp.int32)])
                for j in range(8):
                    s = pl.ds(j * 16, 16)
                    rb[r, s] = rb[r, s] * a16
                return 0
            lax.fori_loop(0, KB, scale, 0)
            pltpu.async_copy(rb, acc_out.at[idxn.at[b]], ss, add=True).wait()

            @pl.when(b + 2 < NB)
            def _():
                pltpu.async_copy(xe_hbm.at[idxe.at[b + 2]], rb, sg)
        return 0
    lax.fori_loop(0, NB // 2, rbody, 0)

    plsc.subcore_barrier()
    pltpu.sync_copy(acc_out.at[pl.ds(sid * 640, 640)],
                    opart_hbm.at[cid, pl.ds(sid * 640, 640)])


def _run_sc5(node3, edge3, alpha, xe):
    fn = pl.kernel(
        _sc5_body,
        out_type=[jax.ShapeDtypeStruct((2, NPAD, C), jnp.float32)],
        mesh=_MESH,
        compiler_params=pltpu.CompilerParams(needs_layout_passes=False),
        scratch_types=[
            pltpu.VMEM((NB, KB), jnp.int32),
            pltpu.VMEM((NB, KB), jnp.int32),
            pltpu.VMEM((NB, KB), jnp.float32),
            pltpu.VMEM((KB, C), jnp.float32),
            pltpu.VMEM((KB, C), jnp.float32),
            pltpu.VMEM_SHARED((NPAD, C), jnp.float32),
            pltpu.SemaphoreType.DMA,
            pltpu.SemaphoreType.DMA,
            pltpu.SemaphoreType.DMA,
            pltpu.SemaphoreType.DMA,
        ],
    )
    return fn(node3, edge3, alpha, xe)[0]


def _tce_body(xp_ref, out_ref):
    out_ref[...] = xp_ref[0] + xp_ref[1]


def _run_tce(xepart):
    return pl.pallas_call(
        _tce_body,
        out_shape=jax.ShapeDtypeStruct((EPAD, C), jnp.float32),
    )(xepart)


def _tcd_body(da0_ref, da1_ref, e0_ref, e1_ref, ra_ref, bn_ref):
    ra_ref[...] = 1.0 / (da0_ref[...] + da1_ref[...] + 1e-16)
    deg = e0_ref[...] + e1_ref[...]
    bn_ref[...] = jnp.where(deg > 0.0,
                            1.0 / jnp.where(deg > 0.0, deg, 1.0), 0.0)


def _run_tcd(dap0, dap1, ep0, ep1):
    return pl.pallas_call(
        _tcd_body,
        out_shape=[
            jax.ShapeDtypeStruct((80, 128), jnp.float32),
            jax.ShapeDtypeStruct((5, 128), jnp.float32),
        ],
    )(dap0.reshape(80, 128), dap1.reshape(80, 128),
      ep0.reshape(5, 128), ep1.reshape(5, 128))


def _tcb_body(esp_ref, ep0_ref, ep1_ref, dp0_ref, dp1_ref, rs_ref, out_ref):
    esums = esp_ref[0] + esp_ref[1]
    edegc = ep0_ref[...] + ep1_ref[...]
    ef = esums[:M]
    sqn = jnp.sum(ef * ef, axis=1, keepdims=True)
    nrm = jnp.sqrt(jnp.maximum(sqn, 1e-24))
    efn = ef / jnp.maximum(nrm, 1e-12)
    ones_c = jnp.ones((M, 1), jnp.float32)
    cos = lax.dot_general(efn, efn, (((1,), (1,)), ((), ())),
                          preferred_element_type=jnp.float32,
                          precision=lax.Precision.HIGHEST)
    g = lax.dot_general(ef, ef, (((1,), (1,)), ((), ())),
                        preferred_element_type=jnp.float32,
                        precision=lax.Precision.HIGHEST)
    sqn_r = lax.dot_general(ones_c, sqn, (((1,), (1,)), ((), ())),
                            preferred_element_type=jnp.float32,
                            precision=lax.Precision.HIGHEST)
    sq = sqn + sqn_r - 2.0 * g
    dist = jnp.where(sq > 0.0, jnp.sqrt(jnp.where(sq > 0.0, sq, 1.0)), 0.0)
    margin = 4.2
    loss_item = cos * dist + (1.0 - cos) * jnp.maximum(margin - dist, 0.0)

    idx640 = lax.broadcasted_iota(jnp.int32, (EPAD, 1), 0)
    present = (edegc > 0.0) & (idx640 < M)
    ne = jnp.max(jnp.where(present, idx640 + 1, 0))
    nef = ne.astype(jnp.float32)
    idx_c = lax.broadcasted_iota(jnp.int32, (M, 1), 0)
    idx_r = lax.broadcasted_iota(jnp.int32, (1, M), 1)
    pmf = ((idx_c < ne).astype(jnp.float32) *
           (idx_r < ne).astype(jnp.float32))
    loss_mean = jnp.sum(loss_item * pmf) / (nef * nef)
    loss_hyper = jnp.abs(loss_mean) / ((nef + 1.0) ** 2)

    d_tot = dp0_ref[...] + dp1_ref[...]
    sum_xi = jnp.sum(d_tot * rs_ref[...])
    sum_xj = jnp.sum(edegc * esums)
    cmean = (sum_xi - sum_xj) / float(NC * C)
    out_ref[0, 0] = jnp.abs(cmean) + loss_hyper


def _run_tcb(espart, ep0, ep1, dp0, dp1, rs_2d):
    return pl.pallas_call(
        _tcb_body,
        out_specs=pl.BlockSpec(memory_space=pltpu.SMEM),
        out_shape=jax.ShapeDtypeStruct((1, 1), jnp.float32),
    )(espart, ep0.reshape(EPAD, 1), ep1.reshape(EPAD, 1),
      dp0.reshape(80, 128), dp1.reshape(80, 128), rs_2d)


def _tcf_body(op_ref, d0_ref, d1_ref, out_ref):
    out_ref[...] = (op_ref[0] + op_ref[1]) * (d0_ref[...] + d1_ref[...])


def _run_tcf(opart, dc0, dc1):
    blk = 1024
    return pl.pallas_call(
        _tcf_body,
        grid=(NPAD // blk,),
        in_specs=[
            pl.BlockSpec((2, blk, C), lambda i: (0, i, 0)),
            pl.BlockSpec((blk, 1), lambda i: (i, 0)),
            pl.BlockSpec((blk, 1), lambda i: (i, 0)),
        ],
        out_specs=pl.BlockSpec((blk, C), lambda i: (i, 0)),
        out_shape=jax.ShapeDtypeStruct((NPAD, C), jnp.float32),
    )(opart, dc0.reshape(NPAD, 1), dc1.reshape(NPAD, 1))


def kernel(x, hyperedge_index, weight, att):
    node = hyperedge_index[0].astype(jnp.int32)
    edge = hyperedge_index[1].astype(jnp.int32)
    npad = NCPAD - NC
    padslots = jnp.arange(npad, dtype=jnp.int32) % 64
    node_p = jnp.concatenate([node, 10016 + padslots])
    edge_p = jnp.concatenate([edge, M + padslots])
    node3 = node_p.reshape(NT, NB, KB)
    edge3 = edge_p.reshape(NT, NB, KB)
    xpad = jnp.pad(x[0], ((0, NPAD - N), (0, 0)))

    att1 = att[0, 0, :C]
    att2 = att[0, 0, C:]
    acat = jnp.stack(
        [att1, att2, jnp.ones((C,), jnp.float32)]
        + [jnp.zeros((C,), jnp.float32)] * 5,
        axis=1)

    xproj, auxT = _run_tca(xpad, weight, acat)
    nd1 = auxT[0]
    nd2 = auxT[1]
    rs_2d = auxT[2].reshape(80, 128)

    dp0, dp1, ep0, ep1, tp0, tp1 = _run_sc1(node3, edge3, nd2)
    d2p0, d2p1 = _run_sc2(node3, edge3, tp0, tp1)
    v2d = _run_tcc(nd1.reshape(80, 128), d2p0, d2p1, tp0, tp1)
    ex, dap0, dap1 = _run_sc3(node3, edge3, nd1, v2d.reshape(NPAD), tp0, tp1)
    ra, bn = _run_tcd(dap0, dap1, ep0, ep1)
    xepart, alpha, espart = _run_sc4(node3, edge3, ex, ra.reshape(NPAD),
                                     bn.reshape(EPAD), xproj)
    xe = _run_tce(xepart)
    opart = _run_sc5(node3, edge3, alpha, xe)

    loss = _run_tcb(espart, ep0, ep1, dp0, dp1, rs_2d)
    out = _run_tcf(opart, dp0, dp1)

    x_updated = out[:N][None]
    return x_updated, loss[0, 0]


# speedup vs baseline: 1.0533x; 1.0533x over previous
import jax
import jax.numpy as jnp
from jax import lax
from jax.experimental import pallas as pl
from jax.experimental.pallas import tpu as pltpu
from jax.experimental.pallas import tpu_sc as plsc

N, C = 10000, 128
NC, M = 160000, 512
NPAD, EPAD = 10240, 640
NT = 32
NB, KB = 40, 128
PB = NB * KB
NCPAD = NT * PB

_MESH = plsc.VectorSubcoreMesh(core_axis_name="c", subcore_axis_name="s")


def _wid():
    return lax.axis_index("s") * 2 + lax.axis_index("c")


def _zero16():
    return jnp.zeros((16,), jnp.float32)


def _zero_1d(ref, nwords):
    def body(i, _):
        ref[pl.ds(i * 16, 16)] = _zero16()
        return 0
    lax.fori_loop(0, nwords // 16, body, 0)


def _zero_2d(ref, nrows):
    def body(i, _):
        for j in range(8):
            ref[i, pl.ds(j * 16, 16)] = _zero16()
        return 0
    lax.fori_loop(0, nrows, body, 0)


def _sum_tables(dst, src1, nwords):
    def body(i, _):
        s = pl.ds(i * 16, 16)
        dst[s] = dst[s] + src1[s]
        return 0
    lax.fori_loop(0, nwords // 16, body, 0)


def _core_write(cid, src, dst0, dst1):
    @pl.when(cid == 0)
    def _():
        pltpu.sync_copy(src, dst0)

    @pl.when(cid == 1)
    def _():
        pltpu.sync_copy(src, dst1)


def _tca_body(x_ref, w_ref, a_ref, xp_ref, aux_ref):
    p = jnp.dot(x_ref[...], w_ref[...], preferred_element_type=jnp.float32,
                precision=lax.Precision.HIGHEST)
    xp_ref[...] = p
    aux_ref[...] = lax.dot_general(
        a_ref[...], p, (((0,), (1,)), ((), ())),
        preferred_element_type=jnp.float32,
        precision=lax.Precision.HIGHEST)


def _run_tca(xpad, weight, acat):
    blk = 1024
    return pl.pallas_call(
        _tca_body,
        grid=(NPAD // blk,),
        in_specs=[
            pl.BlockSpec((blk, C), lambda i: (i, 0)),
            pl.BlockSpec((C, C), lambda i: (0, 0)),
            pl.BlockSpec((C, 8), lambda i: (0, 0)),
        ],
        out_specs=[
            pl.BlockSpec((blk, C), lambda i: (i, 0)),
            pl.BlockSpec((8, blk), lambda i: (0, i)),
        ],
        out_shape=[
            jax.ShapeDtypeStruct((NPAD, C), jnp.float32),
            jax.ShapeDtypeStruct((8, NPAD), jnp.float32),
        ],
    )(xpad, weight, acat)


def _sc1_body(node3_hbm, edge3_hbm, nd2_hbm,
              dp0_hbm, dp1_hbm, ep0_hbm, ep1_hbm, tp0_hbm, tp1_hbm,
              idxn, idxe, val, ones_v, nd2_tab, z1d,
              acc_d, acc_e, acc_t,
              sem_a, sem_b, sem_c):
    cid = lax.axis_index("c")
    sid = lax.axis_index("s")
    wid = _wid()

    _zero_1d(z1d, 640)
    pltpu.sync_copy(z1d, acc_d.at[pl.ds(sid * 640, 640)])

    @pl.when(sid == 0)
    def _():
        pltpu.sync_copy(z1d, acc_e)
        pltpu.sync_copy(z1d, acc_t)

    pltpu.sync_copy(node3_hbm.at[wid], idxn)
    pltpu.sync_copy(edge3_hbm.at[wid], idxe)
    pltpu.sync_copy(nd2_hbm, nd2_tab)

    def gbody(b, _):
        for k in range(8):
            s = pl.ds(k * 16, 16)
            val[b, s] = plsc.load_gather(nd2_tab, [idxn[b, s]])
            ones_v[b, s] = jnp.full((16,), 1.0, jnp.float32)
        return 0
    lax.fori_loop(0, NB, gbody, 0)

    plsc.subcore_barrier()

    def sbody(b, _):
        d1 = pltpu.async_copy(ones_v.at[b], acc_d.at[idxn.at[b]], sem_a,
                              add=True)
        d2 = pltpu.async_copy(ones_v.at[b], acc_e.at[idxe.at[b]], sem_b,
                              add=True)
        d3 = pltpu.async_copy(val.at[b], acc_t.at[idxe.at[b]], sem_c,
                              add=True)
        d1.wait()
        d2.wait()
        d3.wait()
        return 0
    lax.fori_loop(0, NB, sbody, 0)

    plsc.subcore_barrier()

    _core_write(cid, acc_d.at[pl.ds(sid * 640, 640)],
                dp0_hbm.at[pl.ds(sid * 640, 640)],
                dp1_hbm.at[pl.ds(sid * 640, 640)])

    @pl.when(sid == 0)
    def _():
        _core_write(cid, acc_e, ep0_hbm, ep1_hbm)
        _core_write(cid, acc_t, tp0_hbm, tp1_hbm)


def _run_sc1(node3, edge3, nd2):
    fn = pl.kernel(
        _sc1_body,
        out_type=[
            jax.ShapeDtypeStruct((NPAD,), jnp.float32),
            jax.ShapeDtypeStruct((NPAD,), jnp.float32),
            jax.ShapeDtypeStruct((EPAD,), jnp.float32),
            jax.ShapeDtypeStruct((EPAD,), jnp.float32),
            jax.ShapeDtypeStruct((EPAD,), jnp.float32),
            jax.ShapeDtypeStruct((EPAD,), jnp.float32),
        ],
        mesh=_MESH,
        compiler_params=pltpu.CompilerParams(needs_layout_passes=False),
        scratch_types=[
            pltpu.VMEM((NB, KB), jnp.int32),
            pltpu.VMEM((NB, KB), jnp.int32),
            pltpu.VMEM((NB, KB), jnp.float32),
            pltpu.VMEM((NB, KB), jnp.float32),
            pltpu.VMEM((NPAD,), jnp.float32),
            pltpu.VMEM((640,), jnp.float32),
            pltpu.VMEM_SHARED((NPAD,), jnp.float32),
            pltpu.VMEM_SHARED((EPAD,), jnp.float32),
            pltpu.VMEM_SHARED((EPAD,), jnp.float32),
            pltpu.SemaphoreType.DMA,
            pltpu.SemaphoreType.DMA,
            pltpu.SemaphoreType.DMA,
        ],
    )
    return fn(node3, edge3, nd2)


def _sc2_body(node3_hbm, edge3_hbm, tp0_hbm, tp1_hbm,
              d2p0_hbm, d2p1_hbm,
              idxn, idxe, val, t0, t1, z1d, acc,
              sem_a):
    cid = lax.axis_index("c")
    sid = lax.axis_index("s")
    wid = _wid()

    _zero_1d(z1d, 640)
    pltpu.sync_copy(z1d, acc.at[pl.ds(sid * 640, 640)])

    pltpu.sync_copy(node3_hbm.at[wid], idxn)
    pltpu.sync_copy(edge3_hbm.at[wid], idxe)
    pltpu.sync_copy(tp0_hbm, t0)
    pltpu.sync_copy(tp1_hbm, t1)
    _sum_tables(t0, t1, EPAD)

    def mbody(i, m):
        return jnp.maximum(m, t0[pl.ds(i * 16, 16)])
    m16 = lax.fori_loop(0, EPAD // 16, mbody,
                        jnp.full((16,), -3.4e38, jnp.float32))
    gmax = jnp.max(m16)

    def gbody(b, _):
        for k in range(8):
            s = pl.ds(k * 16, 16)
            ed = plsc.load_gather(t0, [idxe[b, s]])
            val[b, s] = jnp.exp(ed - gmax)
        return 0
    lax.fori_loop(0, NB, gbody, 0)

    plsc.subcore_barrier()

    def sbody(b, _):
        pltpu.async_copy(val.at[b], acc.at[idxn.at[b]], sem_a,
                         add=True).wait()
        return 0
    lax.fori_loop(0, NB, sbody, 0)

    plsc.subcore_barrier()
    _core_write(cid, acc.at[pl.ds(sid * 640, 640)],
                d2p0_hbm.at[pl.ds(sid * 640, 640)],
                d2p1_hbm.at[pl.ds(sid * 640, 640)])


def _run_sc2(node3, edge3, tp0, tp1):
    fn = pl.kernel(
        _sc2_body,
        out_type=[
            jax.ShapeDtypeStruct((NPAD,), jnp.float32),
            jax.ShapeDtypeStruct((NPAD,), jnp.float32),
        ],
        mesh=_MESH,
        compiler_params=pltpu.CompilerParams(needs_layout_passes=False),
        scratch_types=[
            pltpu.VMEM((NB, KB), jnp.int32),
            pltpu.VMEM((NB, KB), jnp.int32),
            pltpu.VMEM((NB, KB), jnp.float32),
            pltpu.VMEM((EPAD,), jnp.float32),
            pltpu.VMEM((EPAD,), jnp.float32),
            pltpu.VMEM((640,), jnp.float32),
            pltpu.VMEM_SHARED((NPAD,), jnp.float32),
            pltpu.SemaphoreType.DMA,
        ],
    )
    return fn(node3, edge3, tp0, tp1)


def _tcc_body(nd1_ref, d2a_ref, d2b_ref, ta_ref, tb_ref, v_ref):
    ed2 = ta_ref[...] + tb_ref[...]
    gmax = jnp.max(ed2)
    d2 = d2a_ref[...] + d2b_ref[...]
    z = nd1_ref[...] + gmax + jnp.log(jnp.maximum(d2, 1e-38))
    v_ref[...] = jnp.maximum(z, 0.2 * z)


def _run_tcc(nd1_2d, d2p0, d2p1, tp0, tp1):
    return pl.pallas_call(
        _tcc_body,
        out_shape=jax.ShapeDtypeStruct((80, 128), jnp.float32),
    )(nd1_2d, d2p0.reshape(80, 128), d2p1.reshape(80, 128),
      tp0.reshape(5, 128), tp1.reshape(5, 128))


def _sc3_body(node3_hbm, edge3_hbm, nd1_hbm, v_hbm, tp0_hbm, tp1_hbm,
              ex_hbm, dap0_hbm, dap1_hbm,
              idxn, idxe, exbuf, nd1_tab, v_tab, t0, t1, z1d, acc,
              sem_a):
    cid = lax.axis_index("c")
    sid = lax.axis_index("s")
    wid = _wid()

    _zero_1d(z1d, 640)
    pltpu.sync_copy(z1d, acc.at[pl.ds(sid * 640, 640)])

    pltpu.sync_copy(node3_hbm.at[wid], idxn)
    pltpu.sync_copy(edge3_hbm.at[wid], idxe)
    pltpu.sync_copy(nd1_hbm, nd1_tab)
    pltpu.sync_copy(v_hbm, v_tab)
    pltpu.sync_copy(tp0_hbm, t0)
    pltpu.sync_copy(tp1_hbm, t1)
    _sum_tables(t0, t1, EPAD)

    def gbody(b, _):
        for k in range(8):
            s = pl.ds(k * 16, 16)
            ii = idxn[b, s]
            z = plsc.load_gather(nd1_tab, [ii]) + plsc.load_gather(
                t0, [idxe[b, s]])
            e = jnp.maximum(z, 0.2 * z)
            exbuf[b, s] = jnp.exp(e - plsc.load_gather(v_tab, [ii]))
        return 0
    lax.fori_loop(0, NB, gbody, 0)

    pltpu.sync_copy(exbuf, ex_hbm.at[wid])

    plsc.subcore_barrier()

    def sbody(b, _):
        pltpu.async_copy(exbuf.at[b], acc.at[idxn.at[b]], sem_a,
                         add=True).wait()
        return 0
    lax.fori_loop(0, NB, sbody, 0)

    plsc.subcore_barrier()
    _core_write(cid, acc.at[pl.ds(sid * 640, 640)],
                dap0_hbm.at[pl.ds(sid * 640, 640)],
                dap1_hbm.at[pl.ds(sid * 640, 640)])


def _run_sc3(node3, edge3, nd1, v, tp0, tp1):
    fn = pl.kernel(
        _sc3_body,
        out_type=[
            jax.ShapeDtypeStruct((NT, NB, KB), jnp.float32),
            jax.ShapeDtypeStruct((NPAD,), jnp.float32),
            jax.ShapeDtypeStruct((NPAD,), jnp.float32),
        ],
        mesh=_MESH,
        compiler_params=pltpu.CompilerParams(needs_layout_passes=False),
        scratch_types=[
            pltpu.VMEM((NB, KB), jnp.int32),
            pltpu.VMEM((NB, KB), jnp.int32),
            pltpu.VMEM((NB, KB), jnp.float32),
            pltpu.VMEM((NPAD,), jnp.float32),
            pltpu.VMEM((NPAD,), jnp.float32),
            pltpu.VMEM((EPAD,), jnp.float32),
            pltpu.VMEM((EPAD,), jnp.float32),
            pltpu.VMEM((640,), jnp.float32),
            pltpu.VMEM_SHARED((NPAD,), jnp.float32),
            pltpu.SemaphoreType.DMA,
        ],
    )
    return fn(node3, edge3, nd1, v, tp0, tp1)


def _sc4_body(node3_hbm, edge3_hbm, ex_hbm, ra_hbm, bn_hbm, xproj_hbm,
              xepart_hbm, alpha_hbm, espart_hbm,
              idxn, idxe, ab, ra_tab, bn_tab,
              rb0, rb1, sb0, sb1, acc_xe, acc_es,
              sem_g0, sem_g1, sem_s0, sem_s1, sem_u0, sem_u1):
    cid = lax.axis_index("c")
    sid = lax.axis_index("s")
    wid = _wid()

    _zero_2d(rb0, KB)
    pltpu.sync_copy(rb0.at[pl.ds(0, 40)], acc_xe.at[pl.ds(sid * 40, 40)])
    pltpu.sync_copy(rb0.at[pl.ds(0, 40)], acc_es.at[pl.ds(sid * 40, 40)])

    pltpu.sync_copy(node3_hbm.at[wid], idxn)
    pltpu.sync_copy(edge3_hbm.at[wid], idxe)
    pltpu.sync_copy(ex_hbm.at[wid], ab)
    pltpu.sync_copy(ra_hbm, ra_tab)
    pltpu.sync_copy(bn_hbm, bn_tab)

    def abody(b, _):
        for k in range(8):
            s = pl.ds(k * 16, 16)
            ab[b, s] = ab[b, s] * plsc.load_gather(ra_tab, [idxn[b, s]])
        return 0
    lax.fori_loop(0, NB, abody, 0)
    pltpu.sync_copy(ab, alpha_hbm.at[wid])

    def bbody(b, _):
        for k in range(8):
            s = pl.ds(k * 16, 16)
            ab[b, s] = ab[b, s] * plsc.load_gather(bn_tab, [idxe[b, s]])
        return 0
    lax.fori_loop(0, NB, bbody, 0)

    plsc.subcore_barrier()

    bufs = ((rb0, sb0, sem_g0, sem_s0, sem_u0),
            (rb1, sb1, sem_g1, sem_s1, sem_u1))
    pltpu.async_copy(xproj_hbm.at[idxn.at[0]], rb0, sem_g0)
    pltpu.async_copy(xproj_hbm.at[idxn.at[1]], rb1, sem_g1)

    def rbody(bb, _):
        for p in range(2):
            b = bb * 2 + p
            rb, sb, sg, ss, su = bufs[p]
            pltpu.make_async_copy(xproj_hbm.at[idxn.at[b]], rb, sg).wait()
            un = pltpu.async_copy(rb, acc_es.at[idxe.at[b]], su, add=True)

            def scale(r, _):
                a16 = plsc.load_gather(ab, [jnp.full((16,), b, jnp.int32),
                                            jnp.full((16,), r, jnp.int32)])
                for j in range(8):
                    s = pl.ds(j * 16, 16)
                    sb[r, s] = rb[r, s] * a16
                return 0
            lax.fori_loop(0, KB, scale, 0)
            sc = pltpu.async_copy(sb, acc_xe.at[idxe.at[b]], ss, add=True)
            un.wait()
            sc.wait()

            @pl.when(b + 2 < NB)
            def _():
                pltpu.async_copy(xproj_hbm.at[idxn.at[b + 2]], rb, sg)
        return 0
    lax.fori_loop(0, NB // 2, rbody, 0)

    plsc.subcore_barrier()
    pltpu.sync_copy(acc_xe.at[pl.ds(sid * 40, 40)],
                    xepart_hbm.at[cid, pl.ds(sid * 40, 40)])
    pltpu.sync_copy(acc_es.at[pl.ds(sid * 40, 40)],
                    espart_hbm.at[cid, pl.ds(sid * 40, 40)])


def _run_sc4(node3, edge3, ex, ra, bn, xproj):
    fn = pl.kernel(
        _sc4_body,
        out_type=[
            jax.ShapeDtypeStruct((2, EPAD, C), jnp.float32),
            jax.ShapeDtypeStruct((NT, NB, KB), jnp.float32),
            jax.ShapeDtypeStruct((2, EPAD, C), jnp.float32),
        ],
        mesh=_MESH,
        compiler_params=pltpu.CompilerParams(needs_layout_passes=False),
        scratch_types=[
            pltpu.VMEM((NB, KB), jnp.int32),
            pltpu.VMEM((NB, KB), jnp.int32),
            pltpu.VMEM((NB, KB), jnp.float32),
            pltpu.VMEM((NPAD,), jnp.float32),
            pltpu.VMEM((EPAD,), jnp.float32),
            pltpu.VMEM((KB, C), jnp.float32),
            pltpu.VMEM((KB, C), jnp.float32),
            pltpu.VMEM((KB, C), jnp.float32),
            pltpu.VMEM((KB, C), jnp.float32),
            pltpu.VMEM_SHARED((EPAD, C), jnp.float32),
            pltpu.VMEM_SHARED((EPAD, C), jnp.float32),
            pltpu.SemaphoreType.DMA,
            pltpu.SemaphoreType.DMA,
            pltpu.SemaphoreType.DMA,
            pltpu.SemaphoreType.DMA,
            pltpu.SemaphoreType.DMA,
            pltpu.SemaphoreType.DMA,
        ],
    )
    return fn(node3, edge3, ex, ra, bn, xproj)


def _sc5_body(node3_hbm, edge3_hbm, alpha_hbm, xe_hbm,
              opart_hbm,
              idxn, idxe, ab, rb0, rb1, acc_out,
              sem_g0, sem_g1, sem_s0, sem_s1):
    cid = lax.axis_index("c")
    sid = lax.axis_index("s")
    wid = _wid()

    _zero_2d(rb0, KB)
    for kk in range(640 // KB):
        pltpu.sync_copy(rb0, acc_out.at[pl.ds(sid * 640 + kk * KB, KB)])

    pltpu.sync_copy(node3_hbm.at[wid], idxn)
    pltpu.sync_copy(edge3_hbm.at[wid], idxe)
    pltpu.sync_copy(alpha_hbm.at[wid], ab)

    plsc.subcore_barrier()

    bufs = ((rb0, sem_g0, sem_s0), (rb1, sem_g1, sem_s1))
    pltpu.async_copy(xe_hbm.at[idxe.at[0]], rb0, sem_g0)
    pltpu.async_copy(xe_hbm.at[idxe.at[1]], rb1, sem_g1)

    def rbody(bb, _):
        for p in range(2):
            b = bb * 2 + p
            rb, sg, ss = bufs[p]
            pltpu.make_async_copy(xe_hbm.at[idxe.at[b]], rb, sg).wait()

            def scale(r, _):
                a16 = plsc.load_gather(ab, [jnp.full((16,), b, jnp.int32),
                                            jnp.full((16,), r, jnp.int32)])
                for j in range(8):
                    s = pl.ds(j * 16, 16)
                    rb[r, s] = rb[r, s] * a16
                return 0
            lax.fori_loop(0, KB, scale, 0)
            pltpu.async_copy(rb, acc_out.at[idxn.at[b]], ss, add=True).wait()

            @pl.when(b + 2 < NB)
            def _():
                pltpu.async_copy(xe_hbm.at[idxe.at[b + 2]], rb, sg)
        return 0
    lax.fori_loop(0, NB // 2, rbody, 0)

    plsc.subcore_barrier()
    pltpu.sync_copy(acc_out.at[pl.ds(sid * 640, 640)],
                    opart_hbm.at[cid, pl.ds(sid * 640, 640)])


def _run_sc5(node3, edge3, alpha, xe):
    fn = pl.kernel(
        _sc5_body,
        out_type=[jax.ShapeDtypeStruct((2, NPAD, C), jnp.float32)],
        mesh=_MESH,
        compiler_params=pltpu.CompilerParams(needs_layout_passes=False),
        scratch_types=[
            pltpu.VMEM((NB, KB), jnp.int32),
            pltpu.VMEM((NB, KB), jnp.int32),
            pltpu.VMEM((NB, KB), jnp.float32),
            pltpu.VMEM((KB, C), jnp.float32),
            pltpu.VMEM((KB, C), jnp.float32),
            pltpu.VMEM_SHARED((NPAD, C), jnp.float32),
            pltpu.SemaphoreType.DMA,
            pltpu.SemaphoreType.DMA,
            pltpu.SemaphoreType.DMA,
            pltpu.SemaphoreType.DMA,
        ],
    )
    return fn(node3, edge3, alpha, xe)[0]


def _tce_body(xp_ref, out_ref):
    out_ref[...] = xp_ref[0] + xp_ref[1]


def _run_tce(xepart):
    return pl.pallas_call(
        _tce_body,
        out_shape=jax.ShapeDtypeStruct((EPAD, C), jnp.float32),
    )(xepart)


def _tcd_body(da0_ref, da1_ref, e0_ref, e1_ref, ra_ref, bn_ref):
    ra_ref[...] = 1.0 / (da0_ref[...] + da1_ref[...] + 1e-16)
    deg = e0_ref[...] + e1_ref[...]
    bn_ref[...] = jnp.where(deg > 0.0,
                            1.0 / jnp.where(deg > 0.0, deg, 1.0), 0.0)


def _run_tcd(dap0, dap1, ep0, ep1):
    return pl.pallas_call(
        _tcd_body,
        out_shape=[
            jax.ShapeDtypeStruct((80, 128), jnp.float32),
            jax.ShapeDtypeStruct((5, 128), jnp.float32),
        ],
    )(dap0.reshape(80, 128), dap1.reshape(80, 128),
      ep0.reshape(5, 128), ep1.reshape(5, 128))


def _tcb_body(esp_ref, ep0_ref, ep1_ref, dp0_ref, dp1_ref, rs_ref, out_ref):
    esums = esp_ref[0] + esp_ref[1]
    edegc = ep0_ref[...] + ep1_ref[...]
    ef = esums[:M]
    sqn = jnp.sum(ef * ef, axis=1, keepdims=True)
    nrm = jnp.sqrt(jnp.maximum(sqn, 1e-24))
    efn = ef / jnp.maximum(nrm, 1e-12)
    ones_c = jnp.ones((M, 1), jnp.float32)
    cos = lax.dot_general(efn, efn, (((1,), (1,)), ((), ())),
                          preferred_element_type=jnp.float32,
                          precision=lax.Precision.HIGHEST)
    g = lax.dot_general(ef, ef, (((1,), (1,)), ((), ())),
                        preferred_element_type=jnp.float32,
                        precision=lax.Precision.HIGHEST)
    sqn_r = lax.dot_general(ones_c, sqn, (((1,), (1,)), ((), ())),
                            preferred_element_type=jnp.float32,
                            precision=lax.Precision.HIGHEST)
    sq = sqn + sqn_r - 2.0 * g
    dist = jnp.where(sq > 0.0, jnp.sqrt(jnp.where(sq > 0.0, sq, 1.0)), 0.0)
    margin = 4.2
    loss_item = cos * dist + (1.0 - cos) * jnp.maximum(margin - dist, 0.0)

    idx640 = lax.broadcasted_iota(jnp.int32, (EPAD, 1), 0)
    present = (edegc > 0.0) & (idx640 < M)
    ne = jnp.max(jnp.where(present, idx640 + 1, 0))
    nef = ne.astype(jnp.float32)
    idx_c = lax.broadcasted_iota(jnp.int32, (M, 1), 0)
    idx_r = lax.broadcasted_iota(jnp.int32, (1, M), 1)
    pmf = ((idx_c < ne).astype(jnp.float32) *
           (idx_r < ne).astype(jnp.float32))
    loss_mean = jnp.sum(loss_item * pmf) / (nef * nef)
    loss_hyper = jnp.abs(loss_mean) / ((nef + 1.0) ** 2)

    d_tot = dp0_ref[...] + dp1_ref[...]
    sum_xi = jnp.sum(d_tot * rs_ref[...])
    sum_xj = jnp.sum(edegc * esums)
    cmean = (sum_xi - sum_xj) / float(NC * C)
    out_ref[0, 0] = jnp.abs(cmean) + loss_hyper


def _run_tcb(espart, ep0, ep1, dp0, dp1, rs_2d):
    return pl.pallas_call(
        _tcb_body,
        out_specs=pl.BlockSpec(memory_space=pltpu.SMEM),
        out_shape=jax.ShapeDtypeStruct((1, 1), jnp.float32),
    )(espart, ep0.reshape(EPAD, 1), ep1.reshape(EPAD, 1),
      dp0.reshape(80, 128), dp1.reshape(80, 128), rs_2d)


def _tcf_body(op_ref, d0_ref, d1_ref, out_ref):
    out_ref[...] = (op_ref[0] + op_ref[1]) * (d0_ref[...] + d1_ref[...])


def _run_tcf(opart, dc0, dc1):
    blk = 1024
    return pl.pallas_call(
        _tcf_body,
        grid=(NPAD // blk,),
        in_specs=[
            pl.BlockSpec((2, blk, C), lambda i: (0, i, 0)),
            pl.BlockSpec((blk, 1), lambda i: (i, 0)),
            pl.BlockSpec((blk, 1), lambda i: (i, 0)),
        ],
        out_specs=pl.BlockSpec((blk, C), lambda i: (i, 0)),
        out_shape=jax.ShapeDtypeStruct((NPAD, C), jnp.float32),
    )(opart, dc0.reshape(NPAD, 1), dc1.reshape(NPAD, 1))


def kernel(x, hyperedge_index, weight, att):
    node = hyperedge_index[0].astype(jnp.int32)
    edge = hyperedge_index[1].astype(jnp.int32)
    npad = NCPAD - NC
    padslots = jnp.arange(npad, dtype=jnp.int32) % 64
    node_p = jnp.concatenate([node, 10016 + padslots])
    edge_p = jnp.concatenate([edge, M + padslots])
    node3 = node_p.reshape(NT, NB, KB)
    edge3 = edge_p.reshape(NT, NB, KB)
    xpad = jnp.pad(x[0], ((0, NPAD - N), (0, 0)))

    att1 = att[0, 0, :C]
    att2 = att[0, 0, C:]
    acat = jnp.stack(
        [att1, att2, jnp.ones((C,), jnp.float32)]
        + [jnp.zeros((C,), jnp.float32)] * 5,
        axis=1)

    xproj, auxT = _run_tca(xpad, weight, acat)
    nd1 = auxT[0]
    nd2 = auxT[1]
    rs_2d = auxT[2].reshape(80, 128)

    dp0, dp1, ep0, ep1, tp0, tp1 = _run_sc1(node3, edge3, nd2)
    d2p0, d2p1 = _run_sc2(node3, edge3, tp0, tp1)
    v2d = _run_tcc(nd1.reshape(80, 128), d2p0, d2p1, tp0, tp1)
    ex, dap0, dap1 = _run_sc3(node3, edge3, nd1, v2d.reshape(NPAD), tp0, tp1)
    ra, bn = _run_tcd(dap0, dap1, ep0, ep1)
    xepart, alpha, espart = _run_sc4(node3, edge3, ex, ra.reshape(NPAD),
                                     bn.reshape(EPAD), xproj)
    xe = _run_tce(xepart)
    opart = _run_sc5(node3, edge3, alpha, xe)

    loss = _run_tcb(espart, ep0, ep1, dp0, dp1, rs_2d)
    out = _run_tcf(opart, dp0, dp1)

    x_updated = out[:N][None]
    return x_updated, loss[0, 0]


# speedup vs baseline: 1.0730x; 1.0187x over previous
import jax
import jax.numpy as jnp
from jax import lax
from jax.experimental import pallas as pl
from jax.experimental.pallas import tpu as pltpu
from jax.experimental.pallas import tpu_sc as plsc

N, C = 10000, 128
NC, M = 160000, 512
NPAD, EPAD = 10240, 640
NT = 32
NB, KB = 40, 128
PB = NB * KB
NCPAD = NT * PB

_MESH = plsc.VectorSubcoreMesh(core_axis_name="c", subcore_axis_name="s")


def _wid():
    return lax.axis_index("s") * 2 + lax.axis_index("c")


def _zero16():
    return jnp.zeros((16,), jnp.float32)


def _zero_1d(ref, nwords):
    def body(i, _):
        ref[pl.ds(i * 16, 16)] = _zero16()
        return 0
    lax.fori_loop(0, nwords // 16, body, 0)


def _zero_2d(ref, nrows):
    def body(i, _):
        for j in range(8):
            ref[i, pl.ds(j * 16, 16)] = _zero16()
        return 0
    lax.fori_loop(0, nrows, body, 0)


def _sum_tables(dst, src1, nwords):
    def body(i, _):
        s = pl.ds(i * 16, 16)
        dst[s] = dst[s] + src1[s]
        return 0
    lax.fori_loop(0, nwords // 16, body, 0)


def _core_write(cid, src, dst0, dst1):
    @pl.when(cid == 0)
    def _():
        pltpu.sync_copy(src, dst0)

    @pl.when(cid == 1)
    def _():
        pltpu.sync_copy(src, dst1)


def _tca_body(x_ref, w_ref, a_ref, xp_ref, aux_ref):
    p = jnp.dot(x_ref[...], w_ref[...], preferred_element_type=jnp.float32)
    xp_ref[...] = p
    aux_ref[...] = lax.dot_general(
        a_ref[...], p, (((0,), (1,)), ((), ())),
        preferred_element_type=jnp.float32,
        precision=lax.Precision.HIGHEST)


def _run_tca(xpad, weight, acat):
    blk = 1024
    return pl.pallas_call(
        _tca_body,
        grid=(NPAD // blk,),
        in_specs=[
            pl.BlockSpec((blk, C), lambda i: (i, 0)),
            pl.BlockSpec((C, C), lambda i: (0, 0)),
            pl.BlockSpec((C, 8), lambda i: (0, 0)),
        ],
        out_specs=[
            pl.BlockSpec((blk, C), lambda i: (i, 0)),
            pl.BlockSpec((8, blk), lambda i: (0, i)),
        ],
        out_shape=[
            jax.ShapeDtypeStruct((NPAD, C), jnp.float32),
            jax.ShapeDtypeStruct((8, NPAD), jnp.float32),
        ],
    )(xpad, weight, acat)


def _sc1_body(node3_hbm, edge3_hbm, nd2_hbm,
              dp0_hbm, dp1_hbm, ep0_hbm, ep1_hbm, tp0_hbm, tp1_hbm,
              idxn, idxe, val, ones_v, nd2_tab, z1d,
              acc_d, acc_e, acc_t,
              sem_a, sem_b, sem_c):
    cid = lax.axis_index("c")
    sid = lax.axis_index("s")
    wid = _wid()

    _zero_1d(z1d, 640)
    pltpu.sync_copy(z1d, acc_d.at[pl.ds(sid * 640, 640)])

    @pl.when(sid == 0)
    def _():
        pltpu.sync_copy(z1d, acc_e)
        pltpu.sync_copy(z1d, acc_t)

    pltpu.sync_copy(node3_hbm.at[wid], idxn)
    pltpu.sync_copy(edge3_hbm.at[wid], idxe)
    pltpu.sync_copy(nd2_hbm, nd2_tab)

    def gbody(b, _):
        for k in range(8):
            s = pl.ds(k * 16, 16)
            val[b, s] = plsc.load_gather(nd2_tab, [idxn[b, s]])
            ones_v[b, s] = jnp.full((16,), 1.0, jnp.float32)
        return 0
    lax.fori_loop(0, NB, gbody, 0)

    plsc.subcore_barrier()

    def sbody(b, _):
        d1 = pltpu.async_copy(ones_v.at[b], acc_d.at[idxn.at[b]], sem_a,
                              add=True)
        d2 = pltpu.async_copy(ones_v.at[b], acc_e.at[idxe.at[b]], sem_b,
                              add=True)
        d3 = pltpu.async_copy(val.at[b], acc_t.at[idxe.at[b]], sem_c,
                              add=True)
        d1.wait()
        d2.wait()
        d3.wait()
        return 0
    lax.fori_loop(0, NB, sbody, 0)

    plsc.subcore_barrier()
    plsc.subcore_barrier()

    _core_write(cid, acc_d.at[pl.ds(sid * 640, 640)],
                dp0_hbm.at[pl.ds(sid * 640, 640)],
                dp1_hbm.at[pl.ds(sid * 640, 640)])

    @pl.when(sid == 0)
    def _():
        _core_write(cid, acc_e, ep0_hbm, ep1_hbm)
        _core_write(cid, acc_t, tp0_hbm, tp1_hbm)


def _run_sc1(node3, edge3, nd2):
    fn = pl.kernel(
        _sc1_body,
        out_type=[
            jax.ShapeDtypeStruct((NPAD,), jnp.float32),
            jax.ShapeDtypeStruct((NPAD,), jnp.float32),
            jax.ShapeDtypeStruct((EPAD,), jnp.float32),
            jax.ShapeDtypeStruct((EPAD,), jnp.float32),
            jax.ShapeDtypeStruct((EPAD,), jnp.float32),
            jax.ShapeDtypeStruct((EPAD,), jnp.float32),
        ],
        mesh=_MESH,
        compiler_params=pltpu.CompilerParams(needs_layout_passes=False),
        scratch_types=[
            pltpu.VMEM((NB, KB), jnp.int32),
            pltpu.VMEM((NB, KB), jnp.int32),
            pltpu.VMEM((NB, KB), jnp.float32),
            pltpu.VMEM((NB, KB), jnp.float32),
            pltpu.VMEM((NPAD,), jnp.float32),
            pltpu.VMEM((640,), jnp.float32),
            pltpu.VMEM_SHARED((NPAD,), jnp.float32),
            pltpu.VMEM_SHARED((EPAD,), jnp.float32),
            pltpu.VMEM_SHARED((EPAD,), jnp.float32),
            pltpu.SemaphoreType.DMA,
            pltpu.SemaphoreType.DMA,
            pltpu.SemaphoreType.DMA,
        ],
    )
    return fn(node3, edge3, nd2)


def _sc2_body(node3_hbm, edge3_hbm, tp0_hbm, tp1_hbm,
              d2p0_hbm, d2p1_hbm,
              idxn, idxe, val, t0, t1, z1d, acc,
              sem_a):
    cid = lax.axis_index("c")
    sid = lax.axis_index("s")
    wid = _wid()

    _zero_1d(z1d, 640)
    pltpu.sync_copy(z1d, acc.at[pl.ds(sid * 640, 640)])

    pltpu.sync_copy(node3_hbm.at[wid], idxn)
    pltpu.sync_copy(edge3_hbm.at[wid], idxe)
    pltpu.sync_copy(tp0_hbm, t0)
    pltpu.sync_copy(tp1_hbm, t1)
    _sum_tables(t0, t1, EPAD)

    def mbody(i, m):
        return jnp.maximum(m, t0[pl.ds(i * 16, 16)])
    m16 = lax.fori_loop(0, EPAD // 16, mbody,
                        jnp.full((16,), -3.4e38, jnp.float32))
    gmax = jnp.max(m16)

    def gbody(b, _):
        for k in range(8):
            s = pl.ds(k * 16, 16)
            ed = plsc.load_gather(t0, [idxe[b, s]])
            val[b, s] = jnp.exp(jnp.maximum(ed - gmax, -80.0))
        return 0
    lax.fori_loop(0, NB, gbody, 0)

    plsc.subcore_barrier()

    def sbody(b, _):
        pltpu.async_copy(val.at[b], acc.at[idxn.at[b]], sem_a,
                         add=True).wait()
        return 0
    lax.fori_loop(0, NB, sbody, 0)

    plsc.subcore_barrier()
    plsc.subcore_barrier()
    plsc.subcore_barrier()
    _core_write(cid, acc.at[pl.ds(sid * 640, 640)],
                d2p0_hbm.at[pl.ds(sid * 640, 640)],
                d2p1_hbm.at[pl.ds(sid * 640, 640)])


def _run_sc2(node3, edge3, tp0, tp1):
    fn = pl.kernel(
        _sc2_body,
        out_type=[
            jax.ShapeDtypeStruct((NPAD,), jnp.float32),
            jax.ShapeDtypeStruct((NPAD,), jnp.float32),
        ],
        mesh=_MESH,
        compiler_params=pltpu.CompilerParams(needs_layout_passes=False),
        scratch_types=[
            pltpu.VMEM((NB, KB), jnp.int32),
            pltpu.VMEM((NB, KB), jnp.int32),
            pltpu.VMEM((NB, KB), jnp.float32),
            pltpu.VMEM((EPAD,), jnp.float32),
            pltpu.VMEM((EPAD,), jnp.float32),
            pltpu.VMEM((640,), jnp.float32),
            pltpu.VMEM_SHARED((NPAD,), jnp.float32),
            pltpu.SemaphoreType.DMA,
        ],
    )
    return fn(node3, edge3, tp0, tp1)


def _tcc_body(nd1_ref, d2a_ref, d2b_ref, ta_ref, tb_ref, v_ref):
    ed2 = ta_ref[...] + tb_ref[...]
    gmax = jnp.max(ed2)
    d2 = d2a_ref[...] + d2b_ref[...]
    z = nd1_ref[...] + gmax + jnp.log(jnp.maximum(d2, 1e-38))
    v_ref[...] = jnp.maximum(z, 0.2 * z)


def _run_tcc(nd1_2d, d2p0, d2p1, tp0, tp1):
    return pl.pallas_call(
        _tcc_body,
        out_shape=jax.ShapeDtypeStruct((80, 128), jnp.float32),
    )(nd1_2d, d2p0.reshape(80, 128), d2p1.reshape(80, 128),
      tp0.reshape(5, 128), tp1.reshape(5, 128))


def _sc3_body(node3_hbm, edge3_hbm, nd1_hbm, v_hbm, tp0_hbm, tp1_hbm,
              ex_hbm, dap0_hbm, dap1_hbm,
              idxn, idxe, exbuf, nd1_tab, v_tab, t0, t1, z1d, acc,
              sem_a):
    cid = lax.axis_index("c")
    sid = lax.axis_index("s")
    wid = _wid()

    _zero_1d(z1d, 640)
    pltpu.sync_copy(z1d, acc.at[pl.ds(sid * 640, 640)])

    pltpu.sync_copy(node3_hbm.at[wid], idxn)
    pltpu.sync_copy(edge3_hbm.at[wid], idxe)
    pltpu.sync_copy(nd1_hbm, nd1_tab)
    pltpu.sync_copy(v_hbm, v_tab)
    pltpu.sync_copy(tp0_hbm, t0)
    pltpu.sync_copy(tp1_hbm, t1)
    _sum_tables(t0, t1, EPAD)

    def gbody(b, _):
        for k in range(8):
            s = pl.ds(k * 16, 16)
            ii = idxn[b, s]
            z = plsc.load_gather(nd1_tab, [ii]) + plsc.load_gather(
                t0, [idxe[b, s]])
            e = jnp.maximum(z, 0.2 * z)
            exbuf[b, s] = jnp.exp(
                jnp.maximum(e - plsc.load_gather(v_tab, [ii]), -80.0))
        return 0
    lax.fori_loop(0, NB, gbody, 0)

    pltpu.sync_copy(exbuf, ex_hbm.at[wid])

    plsc.subcore_barrier()

    def sbody(b, _):
        pltpu.async_copy(exbuf.at[b], acc.at[idxn.at[b]], sem_a,
                         add=True).wait()
        return 0
    lax.fori_loop(0, NB, sbody, 0)

    plsc.subcore_barrier()
    plsc.subcore_barrier()
    plsc.subcore_barrier()
    _core_write(cid, acc.at[pl.ds(sid * 640, 640)],
                dap0_hbm.at[pl.ds(sid * 640, 640)],
                dap1_hbm.at[pl.ds(sid * 640, 640)])


def _run_sc3(node3, edge3, nd1, v, tp0, tp1):
    fn = pl.kernel(
        _sc3_body,
        out_type=[
            jax.ShapeDtypeStruct((NT, NB, KB), jnp.float32),
            jax.ShapeDtypeStruct((NPAD,), jnp.float32),
            jax.ShapeDtypeStruct((NPAD,), jnp.float32),
        ],
        mesh=_MESH,
        compiler_params=pltpu.CompilerParams(needs_layout_passes=False),
        scratch_types=[
            pltpu.VMEM((NB, KB), jnp.int32),
            pltpu.VMEM((NB, KB), jnp.int32),
            pltpu.VMEM((NB, KB), jnp.float32),
            pltpu.VMEM((NPAD,), jnp.float32),
            pltpu.VMEM((NPAD,), jnp.float32),
            pltpu.VMEM((EPAD,), jnp.float32),
            pltpu.VMEM((EPAD,), jnp.float32),
            pltpu.VMEM((640,), jnp.float32),
            pltpu.VMEM_SHARED((NPAD,), jnp.float32),
            pltpu.SemaphoreType.DMA,
        ],
    )
    return fn(node3, edge3, nd1, v, tp0, tp1)


def _sc4_body(node3_hbm, edge3_hbm, ex_hbm, ra_hbm, bn_hbm, xproj_hbm,
              xepart_hbm, alpha_hbm, espart_hbm,
              idxn, idxe, ab, ra_tab, bn_tab,
              rb0, rb1, sb0, sb1, acc_xe, acc_es,
              sem_g0, sem_g1, sem_s0, sem_s1, sem_u0, sem_u1):
    cid = lax.axis_index("c")
    sid = lax.axis_index("s")
    wid = _wid()

    _zero_2d(rb0, KB)
    pltpu.sync_copy(rb0.at[pl.ds(0, 40)], acc_xe.at[pl.ds(sid * 40, 40)])
    pltpu.sync_copy(rb0.at[pl.ds(0, 40)], acc_es.at[pl.ds(sid * 40, 40)])

    pltpu.sync_copy(node3_hbm.at[wid], idxn)
    pltpu.sync_copy(edge3_hbm.at[wid], idxe)
    pltpu.sync_copy(ex_hbm.at[wid], ab)
    pltpu.sync_copy(ra_hbm, ra_tab)
    pltpu.sync_copy(bn_hbm, bn_tab)

    def abody(b, _):
        for k in range(8):
            s = pl.ds(k * 16, 16)
            ab[b, s] = ab[b, s] * plsc.load_gather(ra_tab, [idxn[b, s]])
        return 0
    lax.fori_loop(0, NB, abody, 0)
    pltpu.sync_copy(ab, alpha_hbm.at[wid])

    def bbody(b, _):
        for k in range(8):
            s = pl.ds(k * 16, 16)
            ab[b, s] = ab[b, s] * plsc.load_gather(bn_tab, [idxe[b, s]])
        return 0
    lax.fori_loop(0, NB, bbody, 0)

    plsc.subcore_barrier()

    bufs = ((rb0, sb0, sem_g0, sem_s0, sem_u0),
            (rb1, sb1, sem_g1, sem_s1, sem_u1))
    pltpu.async_copy(xproj_hbm.at[idxn.at[0]], rb0, sem_g0)
    pltpu.async_copy(xproj_hbm.at[idxn.at[1]], rb1, sem_g1)

    def rbody(bb, _):
        for p in range(2):
            b = bb * 2 + p
            rb, sb, sg, ss, su = bufs[p]
            pltpu.make_async_copy(xproj_hbm.at[idxn.at[b]], rb, sg).wait()
            un = pltpu.async_copy(rb, acc_es.at[idxe.at[b]], su, add=True)

            def scale(r, _):
                a16 = plsc.load_gather(ab, [jnp.full((16,), b, jnp.int32),
                                            jnp.full((16,), r, jnp.int32)])
                for j in range(8):
                    s = pl.ds(j * 16, 16)
                    sb[r, s] = rb[r, s] * a16
                return 0
            lax.fori_loop(0, KB, scale, 0)
            sc = pltpu.async_copy(sb, acc_xe.at[idxe.at[b]], ss, add=True)
            un.wait()
            sc.wait()

            @pl.when(b + 2 < NB)
            def _():
                pltpu.async_copy(xproj_hbm.at[idxn.at[b + 2]], rb, sg)
        return 0
    lax.fori_loop(0, NB // 2, rbody, 0)

    plsc.subcore_barrier()
    plsc.subcore_barrier()
    pltpu.sync_copy(acc_xe.at[pl.ds(sid * 40, 40)],
                    xepart_hbm.at[cid, pl.ds(sid * 40, 40)])
    pltpu.sync_copy(acc_es.at[pl.ds(sid * 40, 40)],
                    espart_hbm.at[cid, pl.ds(sid * 40, 40)])


def _run_sc4(node3, edge3, ex, ra, bn, xproj):
    fn = pl.kernel(
        _sc4_body,
        out_type=[
            jax.ShapeDtypeStruct((2, EPAD, C), jnp.float32),
            jax.ShapeDtypeStruct((NT, NB, KB), jnp.float32),
            jax.ShapeDtypeStruct((2, EPAD, C), jnp.float32),
        ],
        mesh=_MESH,
        compiler_params=pltpu.CompilerParams(needs_layout_passes=False),
        scratch_types=[
            pltpu.VMEM((NB, KB), jnp.int32),
            pltpu.VMEM((NB, KB), jnp.int32),
            pltpu.VMEM((NB, KB), jnp.float32),
            pltpu.VMEM((NPAD,), jnp.float32),
            pltpu.VMEM((EPAD,), jnp.float32),
            pltpu.VMEM((KB, C), jnp.float32),
            pltpu.VMEM((KB, C), jnp.float32),
            pltpu.VMEM((KB, C), jnp.float32),
            pltpu.VMEM((KB, C), jnp.float32),
            pltpu.VMEM_SHARED((EPAD, C), jnp.float32),
            pltpu.VMEM_SHARED((EPAD, C), jnp.float32),
            pltpu.SemaphoreType.DMA,
            pltpu.SemaphoreType.DMA,
            pltpu.SemaphoreType.DMA,
            pltpu.SemaphoreType.DMA,
            pltpu.SemaphoreType.DMA,
            pltpu.SemaphoreType.DMA,
        ],
    )
    return fn(node3, edge3, ex, ra, bn, xproj)


def _sc5_body(node3_hbm, edge3_hbm, alpha_hbm, xe_hbm,
              opart_hbm,
              idxn, idxe, ab, rb0, rb1, acc_out,
              sem_g0, sem_g1, sem_s0, sem_s1):
    cid = lax.axis_index("c")
    sid = lax.axis_index("s")
    wid = _wid()

    _zero_2d(rb0, KB)
    for kk in range(640 // KB):
        pltpu.sync_copy(rb0, acc_out.at[pl.ds(sid * 640 + kk * KB, KB)])

    pltpu.sync_copy(node3_hbm.at[wid], idxn)
    pltpu.sync_copy(edge3_hbm.at[wid], idxe)
    pltpu.sync_copy(alpha_hbm.at[wid], ab)

    plsc.subcore_barrier()

    bufs = ((rb0, sem_g0, sem_s0), (rb1, sem_g1, sem_s1))
    pltpu.async_copy(xe_hbm.at[idxe.at[0]], rb0, sem_g0)
    pltpu.async_copy(xe_hbm.at[idxe.at[1]], rb1, sem_g1)

    def rbody(bb, _):
        for p in range(2):
            b = bb * 2 + p
            rb, sg, ss = bufs[p]
            pltpu.make_async_copy(xe_hbm.at[idxe.at[b]], rb, sg).wait()

            def scale(r, _):
                a16 = plsc.load_gather(ab, [jnp.full((16,), b, jnp.int32),
                                            jnp.full((16,), r, jnp.int32)])
                for j in range(8):
                    s = pl.ds(j * 16, 16)
                    rb[r, s] = rb[r, s] * a16
                return 0
            lax.fori_loop(0, KB, scale, 0)
            pltpu.async_copy(rb, acc_out.at[idxn.at[b]], ss, add=True).wait()

            @pl.when(b + 2 < NB)
            def _():
                pltpu.async_copy(xe_hbm.at[idxe.at[b + 2]], rb, sg)
        return 0
    lax.fori_loop(0, NB // 2, rbody, 0)

    plsc.subcore_barrier()
    plsc.subcore_barrier()
    pltpu.sync_copy(acc_out.at[pl.ds(sid * 640, 640)],
                    opart_hbm.at[cid, pl.ds(sid * 640, 640)])


def _run_sc5(node3, edge3, alpha, xe):
    fn = pl.kernel(
        _sc5_body,
        out_type=[jax.ShapeDtypeStruct((2, NPAD, C), jnp.float32)],
        mesh=_MESH,
        compiler_params=pltpu.CompilerParams(needs_layout_passes=False),
        scratch_types=[
            pltpu.VMEM((NB, KB), jnp.int32),
            pltpu.VMEM((NB, KB), jnp.int32),
            pltpu.VMEM((NB, KB), jnp.float32),
            pltpu.VMEM((KB, C), jnp.float32),
            pltpu.VMEM((KB, C), jnp.float32),
            pltpu.VMEM_SHARED((NPAD, C), jnp.float32),
            pltpu.SemaphoreType.DMA,
            pltpu.SemaphoreType.DMA,
            pltpu.SemaphoreType.DMA,
            pltpu.SemaphoreType.DMA,
        ],
    )
    return fn(node3, edge3, alpha, xe)[0]


def _tce_body(xp_ref, out_ref):
    out_ref[...] = xp_ref[0] + xp_ref[1]


def _run_tce(xepart):
    return pl.pallas_call(
        _tce_body,
        out_shape=jax.ShapeDtypeStruct((EPAD, C), jnp.float32),
    )(xepart)


def _tcd_body(da0_ref, da1_ref, e0_ref, e1_ref, ra_ref, bn_ref):
    ra_ref[...] = 1.0 / (da0_ref[...] + da1_ref[...] + 1e-16)
    deg = e0_ref[...] + e1_ref[...]
    bn_ref[...] = jnp.where(deg > 0.0,
                            1.0 / jnp.where(deg > 0.0, deg, 1.0), 0.0)


def _run_tcd(dap0, dap1, ep0, ep1):
    return pl.pallas_call(
        _tcd_body,
        out_shape=[
            jax.ShapeDtypeStruct((80, 128), jnp.float32),
            jax.ShapeDtypeStruct((5, 128), jnp.float32),
        ],
    )(dap0.reshape(80, 128), dap1.reshape(80, 128),
      ep0.reshape(5, 128), ep1.reshape(5, 128))


def _tcb_body(esp_ref, ep0_ref, ep1_ref, dp0_ref, dp1_ref, rs_ref, out_ref):
    esums = esp_ref[0] + esp_ref[1]
    edegc = ep0_ref[...] + ep1_ref[...]
    ef = esums[:M]
    sqn = jnp.sum(ef * ef, axis=1, keepdims=True)
    nrm = jnp.sqrt(jnp.maximum(sqn, 1e-24))
    efn = ef / jnp.maximum(nrm, 1e-12)
    ones_c = jnp.ones((M, 1), jnp.float32)
    cos = lax.dot_general(efn, efn, (((1,), (1,)), ((), ())),
                          preferred_element_type=jnp.float32,
                          precision=lax.Precision.HIGHEST)
    g = lax.dot_general(ef, ef, (((1,), (1,)), ((), ())),
                        preferred_element_type=jnp.float32,
                        precision=lax.Precision.HIGHEST)
    sqn_r = lax.dot_general(ones_c, sqn, (((1,), (1,)), ((), ())),
                            preferred_element_type=jnp.float32,
                            precision=lax.Precision.HIGHEST)
    sq = sqn + sqn_r - 2.0 * g
    dist = jnp.where(sq > 0.0, jnp.sqrt(jnp.where(sq > 0.0, sq, 1.0)), 0.0)
    margin = 4.2
    loss_item = cos * dist + (1.0 - cos) * jnp.maximum(margin - dist, 0.0)

    idx640 = lax.broadcasted_iota(jnp.int32, (EPAD, 1), 0)
    present = (edegc > 0.0) & (idx640 < M)
    ne = jnp.max(jnp.where(present, idx640 + 1, 0))
    nef = ne.astype(jnp.float32)
    idx_c = lax.broadcasted_iota(jnp.int32, (M, 1), 0)
    idx_r = lax.broadcasted_iota(jnp.int32, (1, M), 1)
    pmf = ((idx_c < ne).astype(jnp.float32) *
           (idx_r < ne).astype(jnp.float32))
    loss_mean = jnp.sum(loss_item * pmf) / (nef * nef)
    loss_hyper = jnp.abs(loss_mean) / ((nef + 1.0) ** 2)

    d_tot = dp0_ref[...] + dp1_ref[...]
    sum_xi = jnp.sum(d_tot * rs_ref[...])
    sum_xj = jnp.sum(edegc * esums)
    cmean = (sum_xi - sum_xj) / float(NC * C)
    out_ref[0, 0] = jnp.abs(cmean) + loss_hyper


def _run_tcb(espart, ep0, ep1, dp0, dp1, rs_2d):
    return pl.pallas_call(
        _tcb_body,
        out_specs=pl.BlockSpec(memory_space=pltpu.SMEM),
        out_shape=jax.ShapeDtypeStruct((1, 1), jnp.float32),
    )(espart, ep0.reshape(EPAD, 1), ep1.reshape(EPAD, 1),
      dp0.reshape(80, 128), dp1.reshape(80, 128), rs_2d)


def _tcf_body(op_ref, d0_ref, d1_ref, out_ref):
    out_ref[...] = (op_ref[0] + op_ref[1]) * (d0_ref[...] + d1_ref[...])


def _run_tcf(opart, dc0, dc1):
    blk = 1024
    return pl.pallas_call(
        _tcf_body,
        grid=(NPAD // blk,),
        in_specs=[
            pl.BlockSpec((2, blk, C), lambda i: (0, i, 0)),
            pl.BlockSpec((blk, 1), lambda i: (i, 0)),
            pl.BlockSpec((blk, 1), lambda i: (i, 0)),
        ],
        out_specs=pl.BlockSpec((blk, C), lambda i: (i, 0)),
        out_shape=jax.ShapeDtypeStruct((NPAD, C), jnp.float32),
    )(opart, dc0.reshape(NPAD, 1), dc1.reshape(NPAD, 1))


def kernel(x, hyperedge_index, weight, att):
    node = hyperedge_index[0].astype(jnp.int32)
    edge = hyperedge_index[1].astype(jnp.int32)
    npad = NCPAD - NC
    padslots = jnp.arange(npad, dtype=jnp.int32) % 64
    node_p = jnp.concatenate([node, 10016 + padslots])
    edge_p = jnp.concatenate([edge, M + padslots])
    node3 = node_p.reshape(NT, NB, KB)
    edge3 = edge_p.reshape(NT, NB, KB)
    xpad = jnp.pad(x[0], ((0, NPAD - N), (0, 0)))

    att1 = att[0, 0, :C]
    att2 = att[0, 0, C:]
    acat = jnp.stack(
        [att1, att2, jnp.ones((C,), jnp.float32)]
        + [jnp.zeros((C,), jnp.float32)] * 5,
        axis=1)

    xproj, auxT = _run_tca(xpad, weight, acat)
    nd1 = auxT[0]
    nd2 = auxT[1]
    rs_2d = auxT[2].reshape(80, 128)

    dp0, dp1, ep0, ep1, tp0, tp1 = _run_sc1(node3, edge3, nd2)
    d2p0, d2p1 = _run_sc2(node3, edge3, tp0, tp1)
    v2d = _run_tcc(nd1.reshape(80, 128), d2p0, d2p1, tp0, tp1)
    ex, dap0, dap1 = _run_sc3(node3, edge3, nd1, v2d.reshape(NPAD), tp0, tp1)
    ra, bn = _run_tcd(dap0, dap1, ep0, ep1)
    xepart, alpha, espart = _run_sc4(node3, edge3, ex, ra.reshape(NPAD),
                                     bn.reshape(EPAD), xproj)
    xe = _run_tce(xepart)
    opart = _run_sc5(node3, edge3, alpha, xe)

    loss = _run_tcb(espart, ep0, ep1, dp0, dp1, rs_2d)
    out = _run_tcf(opart, dp0, dp1)

    x_updated = out[:N][None]
    return x_updated, loss[0, 0]
